# baseline (speedup 1.0000x reference)
import functools

import jax
import jax.numpy as jnp
from jax import lax
from jax.experimental import pallas as pl
from jax.experimental.pallas import tpu as pltpu
from jax.experimental.pallas import tpu_sc as plsc

VOCAB = 1000000
EMB = 64
MAX_NORM = 1.0
BATCH = 16384

NC = 2
NS = 16
L = 16
NW = NC * NS
BPW = BATCH // NW
NCHUNK = 4
CB = BPW // NCHUNK
NGROUP = BPW // L


def _rsqrt(s):
    i = plsc.bitcast(s, jnp.int32)
    y = plsc.bitcast(jnp.int32(0x5F3759DF) - (i >> 1), jnp.float32)
    for _ in range(3):
        y = y * (1.5 - 0.5 * s * y * y)
    return y


def _sg_body(in_idx_hbm, out_idx_hbm, win_hbm, wout_hbm, o_hbm,
             iidx_v, oidx_v, rin_v, rout_v, res_v, sem):
    wid = lax.axis_index("s") * NC + lax.axis_index("c")
    base = wid * BPW

    pltpu.sync_copy(in_idx_hbm.at[wid], iidx_v)
    pltpu.sync_copy(out_idx_hbm.at[wid], oidx_v)
    copies = []
    for c in range(NCHUNK):
        copies.append(pltpu.async_copy(
            win_hbm.at[iidx_v.at[c]], rin_v.at[pl.ds(c * CB, CB)], sem))
        copies.append(pltpu.async_copy(
            wout_hbm.at[oidx_v.at[c]], rout_v.at[pl.ds(c * CB, CB)], sem))
    for cp in copies:
        cp.wait()

    lanes = lax.iota(jnp.int32, L)

    def group(g, _):
        row = g * L + lanes
        s_in = jnp.zeros((L,), jnp.float32)
        s_out = jnp.zeros((L,), jnp.float32)
        dot = jnp.zeros((L,), jnp.float32)
        for e in range(EMB):
            col = jnp.full((L,), e, jnp.int32)
            a = plsc.load_gather(rin_v, [row, col])
            b = plsc.load_gather(rout_v, [row, col])
            s_in = s_in + a * a
            s_out = s_out + b * b
            dot = dot + a * b
        scale = jnp.minimum(1.0, MAX_NORM * _rsqrt(s_in)) * \
            jnp.minimum(1.0, MAX_NORM * _rsqrt(s_out))
        x = dot * scale
        res_v[pl.ds(g * L, L)] = 1.0 / (1.0 + jnp.exp(-x))
        return _

    lax.fori_loop(0, NGROUP, group, None)
    pltpu.sync_copy(res_v, o_hbm.at[pl.ds(base, BPW)])


@jax.jit
def _skipgram(in_idx, out_idx, w_in, w_out):
    run = functools.partial(
        pl.kernel,
        mesh=plsc.VectorSubcoreMesh(core_axis_name="c", subcore_axis_name="s"),
        out_type=jax.ShapeDtypeStruct((BATCH,), jnp.float32),
        scratch_types=[
            pltpu.VMEM((NCHUNK, CB), jnp.int32),
            pltpu.VMEM((NCHUNK, CB), jnp.int32),
            pltpu.VMEM((BPW, EMB), jnp.float32),
            pltpu.VMEM((BPW, EMB), jnp.float32),
            pltpu.VMEM((BPW,), jnp.float32),
            pltpu.SemaphoreType.DMA,
        ],
        compiler_params=pltpu.CompilerParams(
            needs_layout_passes=False, use_tc_tiling_on_sc=False),
    )(_sg_body)
    return run(in_idx, out_idx, w_in, w_out)


def kernel(inputs, outputs, W_in, W_out):
    in_idx = inputs.reshape(NW, NCHUNK, CB).astype(jnp.int32)
    out_idx = outputs.reshape(NW, NCHUNK, CB).astype(jnp.int32)
    return _skipgram(in_idx, out_idx, W_in, W_out)


# speedup vs baseline: 1.5175x; 1.5175x over previous
import functools

import jax
import jax.numpy as jnp
from jax import lax
from jax.experimental import pallas as pl
from jax.experimental.pallas import tpu as pltpu
from jax.experimental.pallas import tpu_sc as plsc

VOCAB = 1000000
EMB = 64
MAX_NORM = 1.0
BATCH = 16384

NC = 2
NS = 16
L = 16
NW = NC * NS
BPW = BATCH // NW
CB = 32
NCHUNK = BPW // CB
GPC = CB // L
PAD = 128


def _rsqrt(s):
    i = plsc.bitcast(s, jnp.int32)
    y = plsc.bitcast(jnp.int32(0x5F3759DF) - (i >> 1), jnp.float32)
    for _ in range(3):
        y = y * (1.5 - 0.5 * s * y * y)
    return y


def _sg_body(iidx_hbm, oidx_hbm, win_hbm, wout_hbm, o_hbm,
             iidx_v, oidx_v, rin_v, rout_v, res_v, sem):
    wid = lax.axis_index("s") * NC + lax.axis_index("c")
    base = wid * BPW

    pltpu.sync_copy(iidx_hbm.at[wid], iidx_v)
    pltpu.sync_copy(oidx_hbm.at[wid], oidx_v)

    lanes = lax.iota(jnp.int32, L)

    def chunk(c, _):
        copies = []
        for g in range(GPC):
            ivec = iidx_v[pl.ds(c * CB + g * L, L)]
            ovec = oidx_v[pl.ds(c * CB + g * L, L)]
            for j in range(L):
                sel = lanes == j
                t_in = jnp.max(jnp.where(sel, ivec, 0))
                t_out = jnp.max(jnp.where(sel, ovec, 0))
                copies.append(pltpu.async_copy(
                    win_hbm.at[pl.ds(t_in, 1)],
                    rin_v.at[pl.ds(g * L + j, 1)], sem))
                copies.append(pltpu.async_copy(
                    wout_hbm.at[pl.ds(t_out, 1)],
                    rout_v.at[pl.ds(g * L + j, 1)], sem))
        for cp in copies:
            cp.wait()
        for g in range(GPC):
            elem = g * L + lanes
            s_in = jnp.zeros((L,), jnp.float32)
            s_out = jnp.zeros((L,), jnp.float32)
            dot = jnp.zeros((L,), jnp.float32)
            for e in range(EMB):
                col = jnp.full((L,), e, jnp.int32)
                a = plsc.load_gather(rin_v, [elem, col])
                b = plsc.load_gather(rout_v, [elem, col])
                s_in = s_in + a * a
                s_out = s_out + b * b
                dot = dot + a * b
            scale = jnp.minimum(1.0, MAX_NORM * _rsqrt(s_in)) * \
                jnp.minimum(1.0, MAX_NORM * _rsqrt(s_out))
            x = dot * scale
            res_v[pl.ds(c * CB + g * L, L)] = 1.0 / (1.0 + jnp.exp(-x))
        return _

    lax.fori_loop(0, NCHUNK, chunk, None)
    pltpu.sync_copy(res_v, o_hbm.at[pl.ds(base, BPW)])


@jax.jit
def _skipgram(iidx, oidx, w_in, w_out):
    run = functools.partial(
        pl.kernel,
        mesh=plsc.VectorSubcoreMesh(core_axis_name="c", subcore_axis_name="s"),
        out_type=jax.ShapeDtypeStruct((BATCH,), jnp.float32),
        scratch_types=[
            pltpu.VMEM((BPW,), jnp.int32),
            pltpu.VMEM((BPW,), jnp.int32),
            pltpu.VMEM((CB, EMB), jnp.float32),
            pltpu.VMEM((CB, EMB), jnp.float32),
            pltpu.VMEM((BPW,), jnp.float32),
            pltpu.SemaphoreType.DMA,
        ],
        compiler_params=pltpu.CompilerParams(needs_layout_passes=False),
    )(_sg_body)
    return run(iidx, oidx, w_in, w_out)


def kernel(inputs, outputs, W_in, W_out):
    iidx = inputs.reshape(NW, BPW).astype(jnp.int32)
    oidx = outputs.reshape(NW, BPW).astype(jnp.int32)
    return _skipgram(iidx, oidx, W_in, W_out)
